# baseline (speedup 1.0000x reference)
import jax
import jax.numpy as jnp
from jax import lax
from jax.experimental import pallas as pl
from jax.experimental.pallas import tpu as pltpu
from jax.experimental.pallas import tpu_sc as plsc

_B = 16384
_C = 1000
_NC = 2
_NS = 16
_NW = _NC * _NS
_COLS_W = _B // _NW
_CCHUNK = 256
_NCC = _COLS_W // _CCHUNK
_RCHUNK = 200
_NRC = _C // _RCHUNK
_LANES = 16
_NBUF = 2
_NSTEP = _NCC * _NRC
_NOUTER = _NSTEP // _NBUF


def _onehot_body(x_hbm, out_hbm, idx_v, b0, b1, s0, s1, isem):
    wid = lax.axis_index("s") * _NC + lax.axis_index("c")
    base_col = wid * _COLS_W

    idx_copy = pltpu.async_copy(x_hbm.at[pl.ds(base_col, _COLS_W)], idx_v, isem)

    zeros16 = jnp.zeros((_LANES,), jnp.float32)
    ones16 = jnp.ones((_LANES,), jnp.float32)
    lane = lax.iota(jnp.int32, _LANES)
    bufs = (b0, b1)
    sems = (s0, s1)

    def _zero_slab(buf):
        def _zero(r, carry):
            for j in range(_CCHUNK // _LANES):
                buf[r, pl.ds(j * _LANES, _LANES)] = zeros16
            return carry

        lax.fori_loop(0, _RCHUNK, _zero, 0)

    def _dst(s):
        cc = s // _NRC
        rc = s % _NRC
        return out_hbm.at[
            pl.ds(rc * _RCHUNK, _RCHUNK),
            pl.ds(base_col + cc * _CCHUNK, _CCHUNK),
        ]

    def _scatter(buf, s, val):
        cc = s // _NRC
        r0 = (s % _NRC) * _RCHUNK
        for j in range(_CCHUNK // _LANES):
            iv = idx_v[pl.ds(cc * _CCHUNK + j * _LANES, _LANES)]
            m = (iv >= r0) & (iv < r0 + _RCHUNK)
            plsc.store_scatter(buf, [iv - r0, j * _LANES + lane], val, mask=m)

    def _outer(g, carry):
        for k in range(_NBUF):
            s = g * _NBUF + k
            buf = bufs[k]
            sem = sems[k]

            @pl.when(g == 0)
            def _():
                _zero_slab(buf)

            if k == 0:

                @pl.when(g == 0)
                def _():
                    idx_copy.wait()

            @pl.when(g > 0)
            def _():
                pltpu.make_async_copy(buf, _dst(s - _NBUF), sem).wait()
                _scatter(buf, s - _NBUF, zeros16)

            _scatter(buf, s, ones16)
            pltpu.async_copy(buf, _dst(s), sem)
        return carry

    lax.fori_loop(0, _NOUTER, _outer, 0)

    for k in range(_NBUF):
        s = (_NOUTER - 1) * _NBUF + k
        pltpu.make_async_copy(bufs[k], _dst(s), sems[k]).wait()


def kernel(x):
    mesh = plsc.VectorSubcoreMesh(core_axis_name="c", subcore_axis_name="s")
    f = pl.kernel(
        _onehot_body,
        out_type=jax.ShapeDtypeStruct((_C, _B), jnp.float32),
        mesh=mesh,
        compiler_params=pltpu.CompilerParams(
            needs_layout_passes=False,
            skip_device_barrier=True,
            disable_semaphore_checks=True,
        ),
        scratch_types=[
            pltpu.VMEM((_COLS_W,), jnp.int32),
            pltpu.VMEM((_RCHUNK, _CCHUNK), jnp.float32),
            pltpu.VMEM((_RCHUNK, _CCHUNK), jnp.float32),
            pltpu.SemaphoreType.DMA,
            pltpu.SemaphoreType.DMA,
            pltpu.SemaphoreType.DMA,
        ],
    )
    out_t = f(x.reshape(_B))
    return jnp.transpose(out_t)


# speedup vs baseline: 1.0148x; 1.0148x over previous
import jax
import jax.numpy as jnp
from jax import lax
from jax.experimental import pallas as pl
from jax.experimental.pallas import tpu as pltpu
from jax.experimental.pallas import tpu_sc as plsc

_B = 16384
_C = 1000
_NC = 2
_NS = 16
_NW = _NC * _NS
_COLS_W = _B // _NW
_CCHUNK = 128
_NCC = _COLS_W // _CCHUNK
_RCHUNK = 200
_NRC = _C // _RCHUNK
_LANES = 16
_NBUF = 5
_NSTEP = _NCC * _NRC
_NOUTER = _NSTEP // _NBUF


def _onehot_body(x_hbm, out_hbm, idx_v, b0, b1, b2, b3, b4, s0, s1, s2, s3, s4, isem):
    wid = lax.axis_index("s") * _NC + lax.axis_index("c")
    base_col = wid * _COLS_W

    idx_copy = pltpu.async_copy(x_hbm.at[pl.ds(base_col, _COLS_W)], idx_v, isem)

    zeros16 = jnp.zeros((_LANES,), jnp.float32)
    ones16 = jnp.ones((_LANES,), jnp.float32)
    lane = lax.iota(jnp.int32, _LANES)
    bufs = (b0, b1, b2, b3, b4)
    sems = (s0, s1, s2, s3, s4)

    def _zero_slab(buf):
        def _zero(r, carry):
            for j in range(_CCHUNK // _LANES):
                buf[r, pl.ds(j * _LANES, _LANES)] = zeros16
            return carry

        lax.fori_loop(0, _RCHUNK, _zero, 0)

    def _dst(s):
        cc = s // _NRC
        rc = s % _NRC
        return out_hbm.at[
            pl.ds(rc * _RCHUNK, _RCHUNK),
            pl.ds(base_col + cc * _CCHUNK, _CCHUNK),
        ]

    def _scatter(buf, s, val):
        cc = s // _NRC
        r0 = (s % _NRC) * _RCHUNK
        for j in range(_CCHUNK // _LANES):
            iv = idx_v[pl.ds(cc * _CCHUNK + j * _LANES, _LANES)]
            m = (iv >= r0) & (iv < r0 + _RCHUNK)
            plsc.store_scatter(buf, [iv - r0, j * _LANES + lane], val, mask=m)

    def _outer(g, carry):
        for k in range(_NBUF):
            s = g * _NBUF + k
            buf = bufs[k]
            sem = sems[k]

            @pl.when(g == 0)
            def _():
                _zero_slab(buf)

            if k == 0:

                @pl.when(g == 0)
                def _():
                    idx_copy.wait()

            @pl.when(g > 0)
            def _():
                pltpu.make_async_copy(buf, _dst(s - _NBUF), sem).wait()
                _scatter(buf, s - _NBUF, zeros16)

            _scatter(buf, s, ones16)
            pltpu.async_copy(buf, _dst(s), sem)
        return carry

    lax.fori_loop(0, _NOUTER, _outer, 0)

    for k in range(_NBUF):
        s = (_NOUTER - 1) * _NBUF + k
        pltpu.make_async_copy(bufs[k], _dst(s), sems[k]).wait()


def kernel(x):
    mesh = plsc.VectorSubcoreMesh(core_axis_name="c", subcore_axis_name="s")
    f = pl.kernel(
        _onehot_body,
        out_type=jax.ShapeDtypeStruct((_C, _B), jnp.float32),
        mesh=mesh,
        compiler_params=pltpu.CompilerParams(
            needs_layout_passes=False,
            skip_device_barrier=True,
            disable_semaphore_checks=True,
        ),
        scratch_types=[
            pltpu.VMEM((_COLS_W,), jnp.int32),
            pltpu.VMEM((_RCHUNK, _CCHUNK), jnp.float32),
            pltpu.VMEM((_RCHUNK, _CCHUNK), jnp.float32),
            pltpu.VMEM((_RCHUNK, _CCHUNK), jnp.float32),
            pltpu.VMEM((_RCHUNK, _CCHUNK), jnp.float32),
            pltpu.VMEM((_RCHUNK, _CCHUNK), jnp.float32),
            pltpu.SemaphoreType.DMA,
            pltpu.SemaphoreType.DMA,
            pltpu.SemaphoreType.DMA,
            pltpu.SemaphoreType.DMA,
            pltpu.SemaphoreType.DMA,
            pltpu.SemaphoreType.DMA,
        ],
    )
    out_t = f(x.reshape(_B))
    return jnp.transpose(out_t)


# speedup vs baseline: 1.0216x; 1.0067x over previous
import jax
import jax.numpy as jnp
from jax import lax
from jax.experimental import pallas as pl
from jax.experimental.pallas import tpu as pltpu
from jax.experimental.pallas import tpu_sc as plsc

_B = 16384
_C = 1000
_NC = 2
_NS = 16
_NW = _NC * _NS
_COLS_W = _B // _NW
_CCHUNK = 128
_NCC = _COLS_W // _CCHUNK
_RCHUNK = 200
_NRC = _C // _RCHUNK
_LANES = 16
_NBUF = 4
_NSTEP = _NCC * _NRC
_NOUTER = _NSTEP // _NBUF


def _onehot_body(x_hbm, out_hbm, idx_v, b0, b1, b2, b3, s0, s1, s2, s3, isem):
    wid = lax.axis_index("s") * _NC + lax.axis_index("c")
    base_col = wid * _COLS_W

    idx_copy = pltpu.async_copy(x_hbm.at[pl.ds(base_col, _COLS_W)], idx_v, isem)

    zeros16 = jnp.zeros((_LANES,), jnp.float32)
    ones16 = jnp.ones((_LANES,), jnp.float32)
    lane = lax.iota(jnp.int32, _LANES)
    bufs = (b0, b1, b2, b3)
    sems = (s0, s1, s2, s3)

    def _zero_slab(buf):
        def _zero(r, carry):
            for j in range(_CCHUNK // _LANES):
                buf[r, pl.ds(j * _LANES, _LANES)] = zeros16
            return carry

        lax.fori_loop(0, _RCHUNK, _zero, 0)

    def _dst(s):
        cc = s // _NRC
        rc = s % _NRC
        return out_hbm.at[
            pl.ds(rc * _RCHUNK, _RCHUNK),
            pl.ds(base_col + cc * _CCHUNK, _CCHUNK),
        ]

    def _scatter(buf, s, val):
        cc = s // _NRC
        r0 = (s % _NRC) * _RCHUNK
        for j in range(_CCHUNK // _LANES):
            iv = idx_v[pl.ds(cc * _CCHUNK + j * _LANES, _LANES)]
            m = (iv >= r0) & (iv < r0 + _RCHUNK)
            plsc.store_scatter(buf, [iv - r0, j * _LANES + lane], val, mask=m)

    def _outer(g, carry):
        for k in range(_NBUF):
            s = g * _NBUF + k
            buf = bufs[k]
            sem = sems[k]

            @pl.when(g == 0)
            def _():
                _zero_slab(buf)

            if k == 0:

                @pl.when(g == 0)
                def _():
                    idx_copy.wait()

            @pl.when(g > 0)
            def _():
                pltpu.make_async_copy(buf, _dst(s - _NBUF), sem).wait()
                _scatter(buf, s - _NBUF, zeros16)

            _scatter(buf, s, ones16)
            pltpu.async_copy(buf, _dst(s), sem)
        return carry

    lax.fori_loop(0, _NOUTER, _outer, 0)

    for k in range(_NBUF):
        s = (_NOUTER - 1) * _NBUF + k
        pltpu.make_async_copy(bufs[k], _dst(s), sems[k]).wait()


def kernel(x):
    mesh = plsc.VectorSubcoreMesh(core_axis_name="c", subcore_axis_name="s")
    f = pl.kernel(
        _onehot_body,
        out_type=jax.ShapeDtypeStruct((_C, _B), jnp.float32),
        mesh=mesh,
        compiler_params=pltpu.CompilerParams(
            needs_layout_passes=False,
            skip_device_barrier=True,
            disable_semaphore_checks=True,
        ),
        scratch_types=[
            pltpu.VMEM((_COLS_W,), jnp.int32),
            pltpu.VMEM((_RCHUNK, _CCHUNK), jnp.float32),
            pltpu.VMEM((_RCHUNK, _CCHUNK), jnp.float32),
            pltpu.VMEM((_RCHUNK, _CCHUNK), jnp.float32),
            pltpu.VMEM((_RCHUNK, _CCHUNK), jnp.float32),
            pltpu.SemaphoreType.DMA,
            pltpu.SemaphoreType.DMA,
            pltpu.SemaphoreType.DMA,
            pltpu.SemaphoreType.DMA,
            pltpu.SemaphoreType.DMA,
        ],
    )
    out_t = f(x.reshape(_B))
    return jnp.transpose(out_t)
